# baseline (speedup 1.0000x reference)
import functools

import jax
import jax.numpy as jnp
from jax import lax
from jax.experimental import pallas as pl
from jax.experimental.pallas import tpu as pltpu
from jax.experimental.pallas import tpu_sc as plsc

EMBED_DIM = 64
NUM_WORKERS = 32
CHUNK = 128
LANES = 16


def _pos_embed_body(x_hbm, idx_hbm, tab_hbm, out_hbm, idx_v, xb, rows, sem):
    nc = 2
    wid = lax.axis_index("s") * nc + lax.axis_index("c")
    tok_per_worker = x_hbm.shape[0] // NUM_WORKERS
    n_chunks = tok_per_worker // CHUNK
    worker_base = wid * tok_per_worker

    del rows

    @pl.loop(0, n_chunks)
    def _chunk(i):
        base = worker_base + i * CHUNK
        pltpu.sync_copy(idx_hbm.at[pl.ds(base, CHUNK)], idx_v)
        pltpu.sync_copy(x_hbm.at[pl.ds(base, CHUNK)], xb)
        pltpu.async_copy(tab_hbm.at[idx_v], xb, sem, add=True).wait()
        pltpu.sync_copy(xb, out_hbm.at[pl.ds(base, CHUNK)])


@functools.partial(jax.jit, static_argnames=())
def kernel(x, position_indices, embedding):
    b, s, d = x.shape
    n = b * s
    x_flat = x.reshape(n, d)
    idx_flat = position_indices.reshape(n).astype(jnp.int32)

    mesh = plsc.VectorSubcoreMesh(
        core_axis_name="c", subcore_axis_name="s", num_cores=2, num_subcores=16
    )
    out = pl.kernel(
        _pos_embed_body,
        out_type=jax.ShapeDtypeStruct((n, d), x.dtype),
        mesh=mesh,
        scratch_types=[
            pltpu.VMEM((CHUNK,), jnp.int32),
            pltpu.VMEM((CHUNK, d), jnp.float32),
            pltpu.VMEM((CHUNK, d), jnp.float32),
            pltpu.SemaphoreType.DMA,
        ],
        compiler_params=pltpu.CompilerParams(use_tc_tiling_on_sc=False),
    )(x_flat, idx_flat, embedding)
    return out.reshape(b, s, d)


# speedup vs baseline: 1.2794x; 1.2794x over previous
import functools

import jax
import jax.numpy as jnp
from jax import lax
from jax.experimental import pallas as pl
from jax.experimental.pallas import tpu as pltpu
from jax.experimental.pallas import tpu_sc as plsc

NUM_WORKERS = 32
CHUNK = 128
NBUF = 4


def _pos_embed_body(x_hbm, idx_hbm, tab_hbm, out_hbm, idx_v, xb, insem, gsem, osem):
    nc = 2
    wid = lax.axis_index("s") * nc + lax.axis_index("c")
    tok_per_worker = x_hbm.shape[0] // NUM_WORKERS
    n_chunks = tok_per_worker // CHUNK
    worker_base = wid * tok_per_worker

    def issue_in(g):
        b = lax.rem(g, NBUF)
        base = worker_base + g * CHUNK
        pltpu.async_copy(idx_hbm.at[pl.ds(base, CHUNK)], idx_v.at[b], insem.at[b])
        pltpu.async_copy(x_hbm.at[pl.ds(base, CHUNK)], xb.at[b], insem.at[b])

    def issue_gather(g):
        b = lax.rem(g, NBUF)
        base = worker_base + g * CHUNK
        pltpu.make_async_copy(
            idx_hbm.at[pl.ds(base, CHUNK)], idx_v.at[b], insem.at[b]
        ).wait()
        pltpu.make_async_copy(
            x_hbm.at[pl.ds(base, CHUNK)], xb.at[b], insem.at[b]
        ).wait()
        pltpu.async_copy(tab_hbm.at[idx_v.at[b]], xb.at[b], gsem.at[b], add=True)

    def issue_out(g):
        b = lax.rem(g, NBUF)
        base = worker_base + g * CHUNK
        pltpu.make_async_copy(
            tab_hbm.at[idx_v.at[b]], xb.at[b], gsem.at[b]
        ).wait()
        pltpu.async_copy(xb.at[b], out_hbm.at[pl.ds(base, CHUNK)], osem.at[b])

    def wait_out(g):
        b = lax.rem(g, NBUF)
        base = worker_base + g * CHUNK
        pltpu.make_async_copy(
            xb.at[b], out_hbm.at[pl.ds(base, CHUNK)], osem.at[b]
        ).wait()

    @pl.loop(0, n_chunks + 2)
    def _step(t):
        @pl.when(jnp.logical_and(t >= NBUF, t - NBUF < n_chunks))
        def _():
            wait_out(t - NBUF)

        @pl.when(t < n_chunks)
        def _():
            issue_in(t)

        @pl.when(jnp.logical_and(t >= 1, t - 1 < n_chunks))
        def _():
            issue_gather(t - 1)

        @pl.when(t >= 2)
        def _():
            issue_out(t - 2)

    @pl.loop(max(0, n_chunks + 2 - NBUF), n_chunks)
    def _drain(g):
        wait_out(g)


@functools.partial(jax.jit, static_argnames=())
def kernel(x, position_indices, embedding):
    b, s, d = x.shape
    n = b * s
    x_flat = x.reshape(n, d)
    idx_flat = position_indices.reshape(n).astype(jnp.int32)

    mesh = plsc.VectorSubcoreMesh(
        core_axis_name="c", subcore_axis_name="s", num_cores=2, num_subcores=16
    )
    out = pl.kernel(
        _pos_embed_body,
        out_type=jax.ShapeDtypeStruct((n, d), x.dtype),
        mesh=mesh,
        scratch_types=[
            pltpu.VMEM((NBUF, CHUNK), jnp.int32),
            pltpu.VMEM((NBUF, CHUNK, d), jnp.float32),
            pltpu.SemaphoreType.DMA((NBUF,)),
            pltpu.SemaphoreType.DMA((NBUF,)),
            pltpu.SemaphoreType.DMA((NBUF,)),
        ],
        compiler_params=pltpu.CompilerParams(use_tc_tiling_on_sc=False),
    )(x_flat, idx_flat, embedding)
    return out.reshape(b, s, d)


# speedup vs baseline: 1.2847x; 1.0041x over previous
import functools

import jax
import jax.numpy as jnp
from jax import lax
from jax.experimental import pallas as pl
from jax.experimental.pallas import tpu as pltpu
from jax.experimental.pallas import tpu_sc as plsc

NUM_WORKERS = 32
GBLK = 128
GPC = 2
CHUNK = GBLK * GPC
NBUF = 4


def _pos_embed_body(x_hbm, idx_hbm, tab_hbm, out_hbm, idx_v, xb, gsem, xsem, osem):
    nc = 2
    wid = lax.axis_index("s") * nc + lax.axis_index("c")
    tok_per_worker = x_hbm.shape[0] // NUM_WORKERS
    n_chunks = tok_per_worker // CHUNK
    n_gblks = tok_per_worker // GBLK
    worker_base = wid * tok_per_worker

    pltpu.sync_copy(idx_hbm.at[pl.ds(wid * n_gblks, n_gblks)], idx_v)

    def issue_in(g):
        b = lax.rem(g, NBUF)
        base = worker_base + g * CHUNK
        pltpu.async_copy(x_hbm.at[pl.ds(base, CHUNK)], xb.at[b], xsem.at[b])

    def issue_gather(g):
        b = lax.rem(g, NBUF)
        base = worker_base + g * CHUNK
        pltpu.make_async_copy(
            x_hbm.at[pl.ds(base, CHUNK)], xb.at[b], xsem.at[b]
        ).wait()
        for j in range(GPC):
            pltpu.async_copy(
                tab_hbm.at[idx_v.at[g * GPC + j]],
                xb.at[b, pl.ds(j * GBLK, GBLK)],
                gsem.at[b],
                add=True,
            )

    def issue_out(g):
        b = lax.rem(g, NBUF)
        base = worker_base + g * CHUNK
        for j in range(GPC):
            pltpu.make_async_copy(
                tab_hbm.at[idx_v.at[g * GPC + j]],
                xb.at[b, pl.ds(j * GBLK, GBLK)],
                gsem.at[b],
            ).wait()
        pltpu.async_copy(xb.at[b], out_hbm.at[pl.ds(base, CHUNK)], osem.at[b])

    def wait_out(g):
        b = lax.rem(g, NBUF)
        base = worker_base + g * CHUNK
        pltpu.make_async_copy(
            xb.at[b], out_hbm.at[pl.ds(base, CHUNK)], osem.at[b]
        ).wait()

    @pl.loop(0, n_chunks + 2)
    def _step(t):
        @pl.when(jnp.logical_and(t >= NBUF, t - NBUF < n_chunks))
        def _():
            wait_out(t - NBUF)

        @pl.when(t < n_chunks)
        def _():
            issue_in(t)

        @pl.when(jnp.logical_and(t >= 1, t - 1 < n_chunks))
        def _():
            issue_gather(t - 1)

        @pl.when(t >= 2)
        def _():
            issue_out(t - 2)

    @pl.loop(max(0, n_chunks + 2 - NBUF), n_chunks)
    def _drain(g):
        wait_out(g)


@functools.partial(jax.jit, static_argnames=())
def kernel(x, position_indices, embedding):
    b, s, d = x.shape
    n = b * s
    x_flat = x.reshape(n, d)
    idx_2d = position_indices.reshape(n // GBLK, GBLK).astype(jnp.int32)

    mesh = plsc.VectorSubcoreMesh(
        core_axis_name="c", subcore_axis_name="s", num_cores=2, num_subcores=16
    )
    n_gblks_w = n // NUM_WORKERS // GBLK
    out = pl.kernel(
        _pos_embed_body,
        out_type=jax.ShapeDtypeStruct((n, d), x.dtype),
        mesh=mesh,
        scratch_types=[
            pltpu.VMEM((n_gblks_w, GBLK), jnp.int32),
            pltpu.VMEM((NBUF, CHUNK, d), jnp.float32),
            pltpu.SemaphoreType.DMA((NBUF,)),
            pltpu.SemaphoreType.DMA((NBUF,)),
            pltpu.SemaphoreType.DMA((NBUF,)),
        ],
        compiler_params=pltpu.CompilerParams(use_tc_tiling_on_sc=False),
    )(x_flat, idx_2d, embedding)
    return out.reshape(b, s, d)


# speedup vs baseline: 1.3993x; 1.0893x over previous
import functools

import jax
import jax.numpy as jnp
from jax import lax
from jax.experimental import pallas as pl
from jax.experimental.pallas import tpu as pltpu
from jax.experimental.pallas import tpu_sc as plsc

NUM_WORKERS = 32
GBLK = 128
GPC = 2
CHUNK = GBLK * GPC
NBUF = 4


def _pos_embed_body(
    x_hbm, idx_hbm, tab_hbm, out_hbm, tab_sp, idx_v, xb, gsem, xsem, osem
):
    nc = 2
    sid = lax.axis_index("s")
    wid = sid * nc + lax.axis_index("c")
    tok_per_worker = x_hbm.shape[0] // NUM_WORKERS
    n_chunks = tok_per_worker // CHUNK
    n_gblks = tok_per_worker // GBLK
    worker_base = wid * tok_per_worker

    rows_per_sub = tab_hbm.shape[0] // 16
    pltpu.sync_copy(
        tab_hbm.at[pl.ds(sid * rows_per_sub, rows_per_sub)],
        tab_sp.at[pl.ds(sid * rows_per_sub, rows_per_sub)],
    )
    pltpu.sync_copy(idx_hbm.at[pl.ds(wid * n_gblks, n_gblks)], idx_v)
    plsc.subcore_barrier()

    def issue_in(g):
        b = lax.rem(g, NBUF)
        base = worker_base + g * CHUNK
        pltpu.async_copy(x_hbm.at[pl.ds(base, CHUNK)], xb.at[b], xsem.at[b])

    def issue_gather(g):
        b = lax.rem(g, NBUF)
        base = worker_base + g * CHUNK
        pltpu.make_async_copy(
            x_hbm.at[pl.ds(base, CHUNK)], xb.at[b], xsem.at[b]
        ).wait()
        for j in range(GPC):
            pltpu.async_copy(
                tab_sp.at[idx_v.at[g * GPC + j]],
                xb.at[b, pl.ds(j * GBLK, GBLK)],
                gsem.at[b],
                add=True,
            )

    def issue_out(g):
        b = lax.rem(g, NBUF)
        base = worker_base + g * CHUNK
        for j in range(GPC):
            pltpu.make_async_copy(
                tab_sp.at[idx_v.at[g * GPC + j]],
                xb.at[b, pl.ds(j * GBLK, GBLK)],
                gsem.at[b],
            ).wait()
        pltpu.async_copy(xb.at[b], out_hbm.at[pl.ds(base, CHUNK)], osem.at[b])

    def wait_out(g):
        b = lax.rem(g, NBUF)
        base = worker_base + g * CHUNK
        pltpu.make_async_copy(
            xb.at[b], out_hbm.at[pl.ds(base, CHUNK)], osem.at[b]
        ).wait()

    @pl.loop(0, n_chunks + 2)
    def _step(t):
        @pl.when(jnp.logical_and(t >= NBUF, t - NBUF < n_chunks))
        def _():
            wait_out(t - NBUF)

        @pl.when(t < n_chunks)
        def _():
            issue_in(t)

        @pl.when(jnp.logical_and(t >= 1, t - 1 < n_chunks))
        def _():
            issue_gather(t - 1)

        @pl.when(t >= 2)
        def _():
            issue_out(t - 2)

    @pl.loop(max(0, n_chunks + 2 - NBUF), n_chunks)
    def _drain(g):
        wait_out(g)


@functools.partial(jax.jit, static_argnames=())
def kernel(x, position_indices, embedding):
    b, s, d = x.shape
    n = b * s
    x_flat = x.reshape(n, d)
    idx_2d = position_indices.reshape(n // GBLK, GBLK).astype(jnp.int32)

    mesh = plsc.VectorSubcoreMesh(
        core_axis_name="c", subcore_axis_name="s", num_cores=2, num_subcores=16
    )
    n_gblks_w = n // NUM_WORKERS // GBLK
    out = pl.kernel(
        _pos_embed_body,
        out_type=jax.ShapeDtypeStruct((n, d), x.dtype),
        mesh=mesh,
        scratch_types=[
            pltpu.VMEM_SHARED(embedding.shape, jnp.float32),
            pltpu.VMEM((n_gblks_w, GBLK), jnp.int32),
            pltpu.VMEM((NBUF, CHUNK, d), jnp.float32),
            pltpu.SemaphoreType.DMA((NBUF,)),
            pltpu.SemaphoreType.DMA((NBUF,)),
            pltpu.SemaphoreType.DMA((NBUF,)),
        ],
        compiler_params=pltpu.CompilerParams(use_tc_tiling_on_sc=False),
    )(x_flat, idx_2d, embedding)
    return out.reshape(b, s, d)
